# baseline (speedup 1.0000x reference)
import functools

import jax
import jax.numpy as jnp
import numpy as np
from jax import lax
from jax.experimental import pallas as pl
from jax.experimental.pallas import tpu as pltpu
from jax.experimental.pallas import tpu_sc as plsc

NC = 2
NS = 16
NW = NC * NS
CHUNK = 128
F0 = 0.80


def _sc_aggregate(features, src0, dst0, src1, dst1, n_pad, ch0, ch1):
    D = features.shape[1]
    ch = max(ch0, ch1)
    rpt = n_pad // NS

    mesh = plsc.VectorSubcoreMesh(core_axis_name="c", subcore_axis_name="s")

    @functools.partial(
        pl.kernel,
        mesh=mesh,
        out_type=jax.ShapeDtypeStruct((NC, n_pad, D), jnp.float32),
        scratch_types=[
            pltpu.VMEM((ch, CHUNK), jnp.int32),
            pltpu.VMEM((ch, CHUNK), jnp.int32),
            pltpu.VMEM((CHUNK, D), jnp.float32),
            pltpu.VMEM_SHARED((n_pad, D), jnp.float32),
            pltpu.SemaphoreType.DMA,
            pltpu.SemaphoreType.DMA,
        ],
    )
    def agg(feat_hbm, src0_h, dst0_h, src1_h, dst1_h, out_hbm, src_v,
            dst_v, gbuf, acc, sem, sem2):
        c = lax.axis_index("c")
        s = lax.axis_index("s")
        base = s * rpt

        zero = jnp.zeros((16,), jnp.float32)

        def zrow(r, carry):
            for j in range(D // 16):
                gbuf[r, pl.ds(j * 16, 16)] = zero
            return carry

        lax.fori_loop(0, CHUNK, zrow, 0)
        off = 0
        while off < rpt:
            step = min(CHUNK, rpt - off)
            pltpu.sync_copy(gbuf.at[pl.ds(0, step)],
                            acc.at[pl.ds(base + off, step)])
            off += step

        half = CHUNK // 2

        def body(j, carry):
            d0 = pltpu.async_copy(feat_hbm.at[src_v.at[j, pl.ds(0, half)]],
                                  gbuf.at[pl.ds(0, half)], sem)
            d1 = pltpu.async_copy(feat_hbm.at[src_v.at[j, pl.ds(half, half)]],
                                  gbuf.at[pl.ds(half, half)], sem2)
            d0.wait()
            d1.wait()
            pltpu.sync_copy(gbuf, acc.at[dst_v.at[j]], add=True)
            return carry

        @pl.when(c == 0)
        def _():
            pltpu.sync_copy(src0_h.at[s], src_v.at[pl.ds(0, ch0)])
            pltpu.sync_copy(dst0_h.at[s], dst_v.at[pl.ds(0, ch0)])

        @pl.when(c != 0)
        def _():
            pltpu.sync_copy(src1_h.at[s], src_v.at[pl.ds(0, ch1)])
            pltpu.sync_copy(dst1_h.at[s], dst_v.at[pl.ds(0, ch1)])

        plsc.subcore_barrier()

        @pl.when(c == 0)
        def _():
            lax.fori_loop(0, ch0, body, 0)

        @pl.when(c != 0)
        def _():
            lax.fori_loop(0, ch1, body, 0)
        plsc.subcore_barrier()

        pltpu.sync_copy(acc.at[pl.ds(base, rpt)],
                        out_hbm.at[c].at[pl.ds(base, rpt)])

    return agg(features, src0, dst0, src1, dst1)


def _tc_transform(partials, features, W1, W2):
    n, D = features.shape
    outd = W1.shape[1]
    blk = 1000
    assert n % blk == 0

    def body(p0_ref, p1_ref, f_ref, w1_ref, w2_ref, o_ref):
        ns = p0_ref[0] + p1_ref[0]
        f = f_ref[...]
        acc = jnp.dot(ns + f, w1_ref[...], preferred_element_type=jnp.float32)
        acc += jnp.dot(ns * f, w2_ref[...], preferred_element_type=jnp.float32)
        o_ref[...] = jnp.where(acc >= 0, acc, 0.2 * acc)

    p0_spec = pl.BlockSpec((1, blk, D), lambda i: (0, i, 0))
    p1_spec = pl.BlockSpec((1, blk, D), lambda i: (1, i, 0))
    row_spec = pl.BlockSpec((blk, D), lambda i: (i, 0))
    w_spec = pl.BlockSpec((D, outd), lambda i: (0, 0))
    return pl.pallas_call(
        body,
        grid=(n // blk,),
        in_specs=[p0_spec, p1_spec, row_spec, w_spec, w_spec],
        out_specs=pl.BlockSpec((blk, outd), lambda i: (i, 0)),
        out_shape=jax.ShapeDtypeStruct((n, outd), jnp.float32),
    )(partials, partials, features, W1, W2)


def kernel(features, edge_index, W1, W2):
    n, D = features.shape
    E = edge_index.shape[1]
    n_pad = ((n + 1 + NS * 8 - 1) // (NS * 8)) * (NS * 8)

    per_pair = pl.cdiv(E, NS * CHUNK)
    ch0 = int(round(per_pair * F0))
    ch1 = per_pair - ch0

    cap = NS * (ch0 + ch1) * CHUNK
    pad = cap - E
    src = jnp.concatenate([edge_index[0], jnp.zeros((pad,), jnp.int32)])
    dst = jnp.concatenate([edge_index[1], jnp.full((pad,), n, jnp.int32)])
    b = NS * ch0 * CHUNK
    src0 = src[:b].reshape(NS, ch0, CHUNK)
    dst0 = dst[:b].reshape(NS, ch0, CHUNK)
    src1 = src[b:].reshape(NS, ch1, CHUNK)
    dst1 = dst[b:].reshape(NS, ch1, CHUNK)

    partials = _sc_aggregate(features, src0, dst0, src1, dst1,
                             n_pad, ch0, ch1)
    return _tc_transform(partials, features, W1, W2)


# speedup vs baseline: 1.0333x; 1.0333x over previous
import functools

import jax
import jax.numpy as jnp
import numpy as np
from jax import lax
from jax.experimental import pallas as pl
from jax.experimental.pallas import tpu as pltpu
from jax.experimental.pallas import tpu_sc as plsc

NC = 2
NS = 16
NW = NC * NS
CHUNK = 128
F0 = 0.77


def _sc_aggregate(features, src0, dst0, src1, dst1, n_pad, ch0, ch1):
    D = features.shape[1]
    ch = max(ch0, ch1)
    rpt = n_pad // NS

    mesh = plsc.VectorSubcoreMesh(core_axis_name="c", subcore_axis_name="s")

    @functools.partial(
        pl.kernel,
        mesh=mesh,
        out_type=jax.ShapeDtypeStruct((NC, n_pad, D), jnp.float32),
        scratch_types=[
            pltpu.VMEM((ch, CHUNK), jnp.int32),
            pltpu.VMEM((ch, CHUNK), jnp.int32),
            pltpu.VMEM((CHUNK, D), jnp.float32),
            pltpu.VMEM_SHARED((n_pad, D), jnp.float32),
            pltpu.SemaphoreType.DMA,
            pltpu.SemaphoreType.DMA,
        ],
    )
    def agg(feat_hbm, src0_h, dst0_h, src1_h, dst1_h, out_hbm, src_v,
            dst_v, gbuf, acc, sem, sem2):
        c = lax.axis_index("c")
        s = lax.axis_index("s")
        base = s * rpt

        zero = jnp.zeros((16,), jnp.float32)

        def zrow(r, carry):
            for j in range(D // 16):
                gbuf[r, pl.ds(j * 16, 16)] = zero
            return carry

        lax.fori_loop(0, CHUNK, zrow, 0)
        off = 0
        while off < rpt:
            step = min(CHUNK, rpt - off)
            pltpu.sync_copy(gbuf.at[pl.ds(0, step)],
                            acc.at[pl.ds(base + off, step)])
            off += step

        half = CHUNK // 2

        def body(j, carry):
            d0 = pltpu.async_copy(feat_hbm.at[src_v.at[j, pl.ds(0, half)]],
                                  gbuf.at[pl.ds(0, half)], sem)
            d1 = pltpu.async_copy(feat_hbm.at[src_v.at[j, pl.ds(half, half)]],
                                  gbuf.at[pl.ds(half, half)], sem2)
            d0.wait()
            d1.wait()
            pltpu.sync_copy(gbuf, acc.at[dst_v.at[j]], add=True)
            return carry

        @pl.when(c == 0)
        def _():
            pltpu.sync_copy(src0_h.at[s], src_v.at[pl.ds(0, ch0)])
            pltpu.sync_copy(dst0_h.at[s], dst_v.at[pl.ds(0, ch0)])

        @pl.when(c != 0)
        def _():
            pltpu.sync_copy(src1_h.at[s], src_v.at[pl.ds(0, ch1)])
            pltpu.sync_copy(dst1_h.at[s], dst_v.at[pl.ds(0, ch1)])

        plsc.subcore_barrier()

        @pl.when(c == 0)
        def _():
            lax.fori_loop(0, ch0, body, 0)

        @pl.when(c != 0)
        def _():
            lax.fori_loop(0, ch1, body, 0)
        plsc.subcore_barrier()

        pltpu.sync_copy(acc.at[pl.ds(base, rpt)],
                        out_hbm.at[c].at[pl.ds(base, rpt)])

    return agg(features, src0, dst0, src1, dst1)


def _tc_transform(partials, features, W1, W2):
    n, D = features.shape
    outd = W1.shape[1]
    blk = 1000
    assert n % blk == 0

    def body(p0_ref, p1_ref, f_ref, w1_ref, w2_ref, o_ref):
        ns = p0_ref[0] + p1_ref[0]
        f = f_ref[...]
        acc = jnp.dot(ns + f, w1_ref[...], preferred_element_type=jnp.float32)
        acc += jnp.dot(ns * f, w2_ref[...], preferred_element_type=jnp.float32)
        o_ref[...] = jnp.where(acc >= 0, acc, 0.2 * acc)

    p0_spec = pl.BlockSpec((1, blk, D), lambda i: (0, i, 0))
    p1_spec = pl.BlockSpec((1, blk, D), lambda i: (1, i, 0))
    row_spec = pl.BlockSpec((blk, D), lambda i: (i, 0))
    w_spec = pl.BlockSpec((D, outd), lambda i: (0, 0))
    return pl.pallas_call(
        body,
        grid=(n // blk,),
        in_specs=[p0_spec, p1_spec, row_spec, w_spec, w_spec],
        out_specs=pl.BlockSpec((blk, outd), lambda i: (i, 0)),
        out_shape=jax.ShapeDtypeStruct((n, outd), jnp.float32),
    )(partials, partials, features, W1, W2)


def kernel(features, edge_index, W1, W2):
    n, D = features.shape
    E = edge_index.shape[1]
    n_pad = ((n + 1 + NS * 8 - 1) // (NS * 8)) * (NS * 8)

    per_pair = pl.cdiv(E, NS * CHUNK)
    ch0 = int(round(per_pair * F0))
    ch1 = per_pair - ch0

    cap = NS * (ch0 + ch1) * CHUNK
    pad = cap - E
    src = jnp.concatenate([edge_index[0], jnp.zeros((pad,), jnp.int32)])
    dst = jnp.concatenate([edge_index[1], jnp.full((pad,), n, jnp.int32)])
    b = NS * ch0 * CHUNK
    src0 = src[:b].reshape(NS, ch0, CHUNK)
    dst0 = dst[:b].reshape(NS, ch0, CHUNK)
    src1 = src[b:].reshape(NS, ch1, CHUNK)
    dst1 = dst[b:].reshape(NS, ch1, CHUNK)

    partials = _sc_aggregate(features, src0, dst0, src1, dst1,
                             n_pad, ch0, ch1)
    return _tc_transform(partials, features, W1, W2)


# speedup vs baseline: 1.0550x; 1.0210x over previous
import functools

import jax
import jax.numpy as jnp
import numpy as np
from jax import lax
from jax.experimental import pallas as pl
from jax.experimental.pallas import tpu as pltpu
from jax.experimental.pallas import tpu_sc as plsc

NC = 2
NS = 16
NW = NC * NS
CHUNK = 128
F0 = 0.75


def _sc_aggregate(features, src0, dst0, src1, dst1, n_pad, ch0, ch1):
    D = features.shape[1]
    ch = max(ch0, ch1)
    rpt = n_pad // NS

    mesh = plsc.VectorSubcoreMesh(core_axis_name="c", subcore_axis_name="s")

    @functools.partial(
        pl.kernel,
        mesh=mesh,
        out_type=jax.ShapeDtypeStruct((NC, n_pad, D), jnp.float32),
        scratch_types=[
            pltpu.VMEM((ch, CHUNK), jnp.int32),
            pltpu.VMEM((ch, CHUNK), jnp.int32),
            pltpu.VMEM((CHUNK, D), jnp.float32),
            pltpu.VMEM_SHARED((n_pad, D), jnp.float32),
            pltpu.SemaphoreType.DMA,
            pltpu.SemaphoreType.DMA,
        ],
    )
    def agg(feat_hbm, src0_h, dst0_h, src1_h, dst1_h, out_hbm, src_v,
            dst_v, gbuf, acc, sem, sem2):
        c = lax.axis_index("c")
        s = lax.axis_index("s")
        base = s * rpt

        zero = jnp.zeros((16,), jnp.float32)

        def zrow(r, carry):
            for j in range(D // 16):
                gbuf[r, pl.ds(j * 16, 16)] = zero
            return carry

        lax.fori_loop(0, CHUNK, zrow, 0)
        off = 0
        while off < rpt:
            step = min(CHUNK, rpt - off)
            pltpu.sync_copy(gbuf.at[pl.ds(0, step)],
                            acc.at[pl.ds(base + off, step)])
            off += step

        half = CHUNK // 2

        def body(j, carry):
            d0 = pltpu.async_copy(feat_hbm.at[src_v.at[j, pl.ds(0, half)]],
                                  gbuf.at[pl.ds(0, half)], sem)
            d1 = pltpu.async_copy(feat_hbm.at[src_v.at[j, pl.ds(half, half)]],
                                  gbuf.at[pl.ds(half, half)], sem2)
            d0.wait()
            d1.wait()
            pltpu.sync_copy(gbuf, acc.at[dst_v.at[j]], add=True)
            return carry

        @pl.when(c == 0)
        def _():
            pltpu.sync_copy(src0_h.at[s], src_v.at[pl.ds(0, ch0)])
            pltpu.sync_copy(dst0_h.at[s], dst_v.at[pl.ds(0, ch0)])

        @pl.when(c != 0)
        def _():
            pltpu.sync_copy(src1_h.at[s], src_v.at[pl.ds(0, ch1)])
            pltpu.sync_copy(dst1_h.at[s], dst_v.at[pl.ds(0, ch1)])

        plsc.subcore_barrier()

        @pl.when(c == 0)
        def _():
            lax.fori_loop(0, ch0, body, 0)

        @pl.when(c != 0)
        def _():
            lax.fori_loop(0, ch1, body, 0)
        plsc.subcore_barrier()

        pltpu.sync_copy(acc.at[pl.ds(base, rpt)],
                        out_hbm.at[c].at[pl.ds(base, rpt)])

    return agg(features, src0, dst0, src1, dst1)


def _tc_transform(partials, features, W1, W2):
    n, D = features.shape
    outd = W1.shape[1]
    blk = 1000
    assert n % blk == 0

    def body(p0_ref, p1_ref, f_ref, w1_ref, w2_ref, o_ref):
        ns = p0_ref[0] + p1_ref[0]
        f = f_ref[...]
        acc = jnp.dot(ns + f, w1_ref[...], preferred_element_type=jnp.float32)
        acc += jnp.dot(ns * f, w2_ref[...], preferred_element_type=jnp.float32)
        o_ref[...] = jnp.where(acc >= 0, acc, 0.2 * acc)

    p0_spec = pl.BlockSpec((1, blk, D), lambda i: (0, i, 0))
    p1_spec = pl.BlockSpec((1, blk, D), lambda i: (1, i, 0))
    row_spec = pl.BlockSpec((blk, D), lambda i: (i, 0))
    w_spec = pl.BlockSpec((D, outd), lambda i: (0, 0))
    return pl.pallas_call(
        body,
        grid=(n // blk,),
        in_specs=[p0_spec, p1_spec, row_spec, w_spec, w_spec],
        out_specs=pl.BlockSpec((blk, outd), lambda i: (i, 0)),
        out_shape=jax.ShapeDtypeStruct((n, outd), jnp.float32),
    )(partials, partials, features, W1, W2)


def kernel(features, edge_index, W1, W2):
    n, D = features.shape
    E = edge_index.shape[1]
    n_pad = ((n + 1 + NS * 8 - 1) // (NS * 8)) * (NS * 8)

    per_pair = pl.cdiv(E, NS * CHUNK)
    ch0 = int(round(per_pair * F0))
    ch1 = per_pair - ch0

    cap = NS * (ch0 + ch1) * CHUNK
    pad = cap - E
    src = jnp.concatenate([edge_index[0], jnp.zeros((pad,), jnp.int32)])
    dst = jnp.concatenate([edge_index[1], jnp.full((pad,), n, jnp.int32)])
    b = NS * ch0 * CHUNK
    src0 = src[:b].reshape(NS, ch0, CHUNK)
    dst0 = dst[:b].reshape(NS, ch0, CHUNK)
    src1 = src[b:].reshape(NS, ch1, CHUNK)
    dst1 = dst[b:].reshape(NS, ch1, CHUNK)

    partials = _sc_aggregate(features, src0, dst0, src1, dst1,
                             n_pad, ch0, ch1)
    return _tc_transform(partials, features, W1, W2)
